# Initial kernel scaffold; baseline (speedup 1.0000x reference)
import jax
import jax.numpy as jnp
from jax.experimental import pallas as pl


def kernel(x, w1, bn1_gamma, bn1_beta, bn1_mean, bn1_var, w2, bn2_gamma, bn2_beta, bn2_mean, bn2_var):
    raise NotImplementedError("write your pallas kernel here")


# speedup vs baseline: 1.9050x; 1.9050x over previous
import functools

import jax
import jax.numpy as jnp
from jax import lax
from jax.experimental import pallas as pl
from jax.experimental.pallas import tpu as pltpu


def _fold_bn(gamma, beta, mean, var, eps=1e-5):
    scale = gamma.astype(jnp.float32) / jnp.sqrt(var.astype(jnp.float32) + eps)
    bias = beta.astype(jnp.float32) - mean.astype(jnp.float32) * scale
    return scale.reshape(1, -1), bias.reshape(1, -1)


def _conv_w_to_k(w_oihw):
    cout, cin = w_oihw.shape[0], w_oihw.shape[1]
    return (jnp.transpose(w_oihw, (2, 3, 1, 0))
            .reshape(9 * cin, cout).astype(jnp.bfloat16))


def _block_kernel(x_ref, w1_ref, s1_ref, b1_ref, w2_ref, s2_ref, b2_ref,
                  out_ref, x9_ref, y9_ref, *, nb, C, H, W, base, rows):
    f32 = jnp.float32
    P = H * W
    M = nb * P

    ztop = base + W + 1
    ztop += (-ztop) % 8
    zbot = ((base - (W + 1) + P) // 8) * 8

    @pl.when(pl.program_id(0) == 0)
    def _zero_halos():
        for ref in (x9_ref, y9_ref):
            ref[:, :ztop, :] = jnp.zeros((nb, ztop, 9 * C), jnp.bfloat16)
            ref[:, zbot:, :] = jnp.zeros((nb, rows - zbot, 9 * C),
                                         jnp.bfloat16)

    col = lax.broadcasted_iota(jnp.int32, (P, 1), 0) % W
    not_first = (col != 0)
    not_last = (col != W - 1)

    def stage(ref, i, t):
        zeros = jnp.zeros_like(t)
        variants = [jnp.where(not_last, t, zeros), t,
                    jnp.where(not_first, t, zeros)]
        for kh in range(3):
            for kw in range(3):
                d = (kh - 1) * W + kw - 1
                k = kh * 3 + kw
                ref[i, base - d:base - d + P, k * C:(k + 1) * C] = variants[kw]

    def conv(ref, i, w_ref):
        win = ref[i, base:base + P, :]
        return jnp.dot(win, w_ref[...], preferred_element_type=f32)

    for i in range(nb):
        xt = jnp.swapaxes(x_ref[i].astype(jnp.bfloat16), 0, 1)
        stage(x9_ref, i, xt)
        y1 = conv(x9_ref, i, w1_ref)
        y1 = jnp.maximum(y1 * s1_ref[...] + b1_ref[...], 0.0)
        stage(y9_ref, i, y1.astype(jnp.bfloat16))
        y2 = conv(y9_ref, i, w2_ref)
        y2 = y2 * s2_ref[...] + b2_ref[...]
        y2t = jnp.swapaxes(y2, 0, 1)
        out_ref[i] = jnp.maximum(y2t + x_ref[i], 0.0)


def kernel(x, w1, bn1_gamma, bn1_beta, bn1_mean, bn1_var,
           w2, bn2_gamma, bn2_beta, bn2_mean, bn2_var):
    N, C, H, W = x.shape
    P = H * W

    w1k = _conv_w_to_k(w1)
    w2k = _conv_w_to_k(w2)
    s1, b1 = _fold_bn(bn1_gamma, bn1_beta, bn1_mean, bn1_var)
    s2, b2 = _fold_bn(bn2_gamma, bn2_beta, bn2_mean, bn2_var)
    consts = [w1k, s1, b1, w2k, s2, b2]

    nb = next(cand for cand in (4, 2, 1) if N % cand == 0)
    grid = (N // nb,)

    base = W + 1
    base += (-base) % 8
    rows = base + (W + 1) + P
    rows += (-rows) % 8

    body = functools.partial(_block_kernel, nb=nb, C=C, H=H, W=W,
                             base=base, rows=rows)

    flops = 2 * N * P * C * (9 * C + 9 * C)
    bytes_accessed = int(2 * N * C * P * 4
                         + sum(int(a.size) * a.dtype.itemsize for a in consts))
    cost = pl.CostEstimate(flops=int(flops), transcendentals=0,
                           bytes_accessed=bytes_accessed)

    const_specs = [
        pl.BlockSpec(a.shape, lambda n, _nd=a.ndim: (0,) * _nd)
        for a in consts
    ]

    out_flat = pl.pallas_call(
        body,
        out_shape=jax.ShapeDtypeStruct((N, C, P), jnp.float32),
        grid=grid,
        in_specs=[pl.BlockSpec((nb, C, P), lambda n: (n, 0, 0))] + const_specs,
        out_specs=pl.BlockSpec((nb, C, P), lambda n: (n, 0, 0)),
        scratch_shapes=[
            pltpu.VMEM((nb, rows, 9 * C), jnp.bfloat16),
            pltpu.VMEM((nb, rows, 9 * C), jnp.bfloat16),
        ],
        compiler_params=pltpu.CompilerParams(
            dimension_semantics=("parallel",),
            vmem_limit_bytes=48 * 1024 * 1024),
        cost_estimate=cost,
    )(x.reshape(N, C, P), *consts)

    return out_flat.reshape(N, C, H, W)


# speedup vs baseline: 1.9973x; 1.0484x over previous
import functools

import jax
import jax.numpy as jnp
from jax import lax
from jax.experimental import pallas as pl
from jax.experimental.pallas import tpu as pltpu


def _fold_bn(gamma, beta, mean, var, eps=1e-5):
    scale = gamma.astype(jnp.float32) / jnp.sqrt(var.astype(jnp.float32) + eps)
    bias = beta.astype(jnp.float32) - mean.astype(jnp.float32) * scale
    return scale.reshape(1, -1), bias.reshape(1, -1)


def _conv_w_to_k(w_oihw):
    cout, cin = w_oihw.shape[0], w_oihw.shape[1]
    return (jnp.transpose(w_oihw, (2, 3, 1, 0))
            .reshape(9 * cin, cout).astype(jnp.bfloat16))


def _block_kernel(x_ref, w1_ref, s1_ref, b1_ref, w2_ref, s2_ref, b2_ref,
                  out_ref, *scratch, nb, C, H, W, base, rows):
    f32 = jnp.float32
    P = H * W
    x9_refs, y9_refs = scratch[:nb], scratch[nb:]
    ny = len(y9_refs)

    ztop = base + W + 1
    ztop += (-ztop) % 8
    zbot = ((base - (W + 1) + P) // 8) * 8

    @pl.when(pl.program_id(0) == 0)
    def _zero_halos():
        for ref in scratch:
            ref[:ztop, :] = jnp.zeros((ztop, 9 * C), jnp.bfloat16)
            ref[zbot:, :] = jnp.zeros((rows - zbot, 9 * C), jnp.bfloat16)

    col = lax.broadcasted_iota(jnp.int32, (P, 1), 0) % W
    not_first = (col != 0)
    not_last = (col != W - 1)

    def stage(ref, t):
        zeros = jnp.zeros_like(t)
        variants = [jnp.where(not_last, t, zeros), t,
                    jnp.where(not_first, t, zeros)]
        for kh in range(3):
            for kw in range(3):
                d = (kh - 1) * W + kw - 1
                k = kh * 3 + kw
                ref[base - d:base - d + P, k * C:(k + 1) * C] = variants[kw]

    def conv(ref, w_ref):
        win = ref[base:base + P, :]
        return jnp.dot(win, w_ref[...], preferred_element_type=f32)

    for i in range(nb):
        xt = jnp.swapaxes(x_ref[i].astype(jnp.bfloat16), 0, 1)
        stage(x9_refs[i], xt)
        y1 = conv(x9_refs[i], w1_ref)
        y1 = jnp.maximum(y1 * s1_ref[...] + b1_ref[...], 0.0)
        stage(y9_refs[i % ny], y1.astype(jnp.bfloat16))
        y2 = conv(y9_refs[i % ny], w2_ref)
        y2 = y2 * s2_ref[...] + b2_ref[...]
        y2t = jnp.swapaxes(y2, 0, 1)
        out_ref[i] = jnp.maximum(y2t + x_ref[i], 0.0)


def kernel(x, w1, bn1_gamma, bn1_beta, bn1_mean, bn1_var,
           w2, bn2_gamma, bn2_beta, bn2_mean, bn2_var):
    N, C, H, W = x.shape
    P = H * W

    s1, b1 = _fold_bn(bn1_gamma, bn1_beta, bn1_mean, bn1_var)
    s2, b2 = _fold_bn(bn2_gamma, bn2_beta, bn2_mean, bn2_var)
    w1k = _conv_w_to_k(w1)
    w2k = _conv_w_to_k(w2)
    consts = [w1k, s1, b1, w2k, s2, b2]

    nb = next(cand for cand in (8, 4, 2, 1) if N % cand == 0)
    ny = min(nb, 3)
    grid = (N // nb,)

    base = W + 1
    base += (-base) % 8
    rows = base + (W + 1) + P
    rows += (-rows) % 8

    body = functools.partial(_block_kernel, nb=nb, C=C, H=H, W=W,
                             base=base, rows=rows)

    flops = 2 * N * P * C * (9 * C + 9 * C)
    bytes_accessed = int(2 * N * C * P * 4
                         + sum(int(a.size) * a.dtype.itemsize for a in consts))
    cost = pl.CostEstimate(flops=int(flops), transcendentals=0,
                           bytes_accessed=bytes_accessed)

    const_specs = [
        pl.BlockSpec(a.shape, lambda n, _nd=a.ndim: (0,) * _nd)
        for a in consts
    ]

    out_flat = pl.pallas_call(
        body,
        out_shape=jax.ShapeDtypeStruct((N, C, P), jnp.float32),
        grid=grid,
        in_specs=[pl.BlockSpec((nb, C, P), lambda n: (n, 0, 0))] + const_specs,
        out_specs=pl.BlockSpec((nb, C, P), lambda n: (n, 0, 0)),
        scratch_shapes=[pltpu.VMEM((rows, 9 * C), jnp.bfloat16)
                        for _ in range(nb + ny)],
        compiler_params=pltpu.CompilerParams(
            dimension_semantics=("parallel",),
            vmem_limit_bytes=48 * 1024 * 1024),
        cost_estimate=cost,
    )(x.reshape(N, C, P), *consts)

    return out_flat.reshape(N, C, H, W)
